# Initial kernel scaffold; baseline (speedup 1.0000x reference)
import jax
import jax.numpy as jnp
from jax.experimental import pallas as pl


def kernel(probas, labels):
    raise NotImplementedError("write your pallas kernel here")


# speedup vs baseline: 20.9182x; 20.9182x over previous
import functools

import jax
import jax.numpy as jnp
from jax import lax
from jax.experimental import pallas as pl
from jax.experimental.pallas import tpu as pltpu
from jax.experimental.pallas import tpu_sc as plsc

B = 1024
NC, NS = 2, 16
NW = NC * NS
CHUNK_ROWS = 512


def _hist_kernel(C, elems_per_w, chunk_f):
    hsize = C * 2 * B
    nchunks = elems_per_w // chunk_f
    bf = jnp.float32(B)

    mesh = plsc.VectorSubcoreMesh(
        core_axis_name="c", subcore_axis_name="s", num_cores=NC, num_subcores=NS
    )

    @functools.partial(
        pl.kernel,
        mesh=mesh,
        compiler_params=pltpu.CompilerParams(needs_layout_passes=False),
        out_type=jax.ShapeDtypeStruct((NW, hsize), jnp.float32),
        scratch_types=[
            pltpu.VMEM((chunk_f,), jnp.float32),
            pltpu.VMEM((chunk_f,), jnp.float32),
            pltpu.VMEM((chunk_f,), jnp.int32),
            pltpu.VMEM((chunk_f,), jnp.int32),
            pltpu.VMEM((chunk_f,), jnp.int32),
            pltpu.VMEM((hsize,), jnp.float32),
            pltpu.SemaphoreType.DMA,
            pltpu.SemaphoreType.DMA,
            pltpu.SemaphoreType.DMA,
            pltpu.SemaphoreType.DMA,
        ],
    )
    def body(probas_hbm, labels_hbm, cls_hbm, out_hbm,
             pbuf0, pbuf1, lbuf0, lbuf1, cbuf, hist, sp0, sp1, sl0, sl1):
        pbufs = (pbuf0, pbuf1)
        lbufs = (lbuf0, lbuf1)
        wid = lax.axis_index("s") * NC + lax.axis_index("c")
        base = wid * elems_per_w

        zeros16 = jnp.zeros((16,), jnp.float32)

        def zbody(i, carry):
            hist[pl.ds(i * 16, 16)] = zeros16
            return carry

        lax.fori_loop(0, hsize // 16, zbody, 0)

        pltpu.sync_copy(cls_hbm, cbuf)

        psems = (sp0, sp1)
        lsems = (sl0, sl1)

        def start(g, slot):
            off = base + g * chunk_f
            pltpu.async_copy(probas_hbm.at[pl.ds(off, chunk_f)],
                             pbufs[slot], psems[slot])
            pltpu.async_copy(labels_hbm.at[pl.ds(off, chunk_f)],
                             lbufs[slot], lsems[slot])

        def wait(slot):
            pltpu.make_async_copy(probas_hbm.at[pl.ds(0, chunk_f)],
                                  pbufs[slot], psems[slot]).wait()
            pltpu.make_async_copy(labels_hbm.at[pl.ds(0, chunk_f)],
                                  lbufs[slot], lsems[slot]).wait()

        ones = jnp.full((16,), 1.0, jnp.float32)

        def compute(slot):
            pb = pbufs[slot]
            lb = lbufs[slot]

            def ibody(i, carry):
                o = i * 16
                vp = pb[pl.ds(o, 16)]
                vl = lb[pl.ds(o, 16)]
                vc = cbuf[pl.ds(o, 16)]
                fg = vl.astype(jnp.float32)
                e = jnp.abs(fg - vp)
                bi = jnp.minimum((e * bf).astype(jnp.int32), B - 1)
                idx = vc + vl * B + bi
                plsc.addupdate_scatter(hist, [idx], ones)
                return carry

            lax.fori_loop(0, chunk_f // 16, ibody, 0, unroll=4)

        start(0, 0)

        def pair(j, carry):
            g0 = 2 * j
            start(g0 + 1, 1)
            wait(0)
            compute(0)

            @pl.when(j < nchunks // 2 - 1)
            def _():
                start(g0 + 2, 0)

            wait(1)
            compute(1)
            return carry

        lax.fori_loop(0, nchunks // 2, pair, 0)

        pltpu.sync_copy(hist, out_hbm.at[wid])

    return body


def _sweep_kernel(C):
    def body(h_ref, o_ref):
        H = jnp.sum(h_ref[...], axis=0)
        m = H[:, :B] + H[:, B:]
        p = H[:, B:]
        r = lax.broadcasted_iota(jnp.int32, (B, B), 0)
        c = lax.broadcasted_iota(jnp.int32, (B, B), 1)
        tri = (r <= c).astype(jnp.float32)
        Sm = jnp.dot(m, tri, preferred_element_type=jnp.float32)
        Sp = jnp.dot(p, tri, preferred_element_type=jnp.float32)
        Mtot = Sm[:, B - 1:B]
        G = Sp[:, B - 1:B]
        Ns = Mtot - Sm
        Ne = Ns + m
        Ks = G - Sp
        Ke = Ks + p
        Js = 1.0 - (G - Ks) / jnp.maximum(G + Ns - Ks, 1.0)
        Je = 1.0 - (G - Ke) / jnp.maximum(G + Ne - Ke, 1.0)
        emid = (lax.broadcasted_iota(jnp.int32, (C, B), 1).astype(jnp.float32)
                + 0.5) * (1.0 / B)
        losses = jnp.sum(emid * (Je - Js), axis=1, keepdims=True)
        pres = (G > 0).astype(jnp.float32)
        num = jnp.sum(losses * pres)
        den = jnp.maximum(jnp.sum(pres), 1.0)
        o_ref[...] = (num / den)[None, None]

    return body


def kernel(probas, labels):
    Pn, C = probas.shape
    chunk_f = CHUNK_ROWS * C
    elems_per_w = (Pn * C) // NW

    pf = probas.reshape(-1)
    lf = labels.reshape(-1)
    cls = jnp.tile(jnp.arange(C, dtype=jnp.int32) * (2 * B), CHUNK_ROWS)

    hist = _hist_kernel(C, elems_per_w, chunk_f)(pf, lf, cls)

    out = pl.pallas_call(
        _sweep_kernel(C),
        out_shape=jax.ShapeDtypeStruct((1, 1), jnp.float32),
    )(hist.reshape(NW, C, 2 * B))
    return out[0, 0]
